# baseline (speedup 1.0000x reference)
import jax
import jax.numpy as jnp
from jax import lax
from jax.experimental import pallas as pl
from jax.experimental.pallas import tpu as pltpu
from jax.experimental.pallas import tpu_sc as plsc

_VOC = 1000
_EMB = 64
_C = 4
_K = 20
_NSC = _K + 1
_NW = 32
_L = 16
_QB = 128
_LP = (3.507552052950621e-06, 0.9997924357286277, -0.4969779111678143,
       0.31459053537160714, -0.18878267362211323, 0.08172680837613401,
       -0.01720806112146555)


def _tc_gram(u_ref, g_ref):
    U = u_ref[...]
    g_ref[...] = lax.dot_general(
        U, U, (((1,), (1,)), ((), ())),
        preferred_element_type=jnp.float32,
        precision=jax.lax.Precision.HIGHEST)


def _sc_vals(x_hbm, t_hbm, n_hbm, g_hbm, out_hbm,
             g_sp, x_v, t_v, n_v, lst_v, val_v, s_v, acc_v, sem, gsem):
    bpw = t_v.shape[0]
    nq = bpw // _QB
    cid = lax.axis_index("c")
    sid = lax.axis_index("s")
    w = sid * 2 + cid
    base = w * bpw

    gdesc = pltpu.make_async_copy(g_hbm, g_sp, gsem)

    @pl.when(sid == 0)
    def _():
        pltpu.async_copy(g_hbm, g_sp, gsem)

    pltpu.sync_copy(x_hbm.at[pl.ds(base * _C, bpw * _C)], x_v)
    pltpu.sync_copy(t_hbm.at[pl.ds(base, bpw)], t_v)
    pltpu.sync_copy(n_hbm.at[pl.ds(base * _K, bpw * _K)], n_v)

    iota = lax.iota(jnp.int32, _L)
    zeros = jnp.zeros((_L,), jnp.float32)
    acc_v[...] = zeros

    def zero_body(g, carry):
        for t in range(_NSC):
            s_v[t, pl.ds(g * _L, _L)] = zeros
        return carry

    lax.fori_loop(0, bpw // _L, zero_body, 0)

    def build(c, q, p):
        def body(g, carry):
            b0 = q * _QB + g * _L
            bvec = b0 + iota
            xc = plsc.load_gather(x_v, [bvec * _C + c])
            for t in range(_NSC):
                if t == 0:
                    tv = t_v[pl.ds(b0, _L)] * _VOC
                else:
                    tv = plsc.load_gather(n_v, [bvec * _K + (t - 1)]) * _VOC
                lst_v[p * _NSC + t, pl.ds(g * _L, _L)] = tv + xc
            return carry
        lax.fori_loop(0, _QB // _L, body, 0)

    def fire(p):
        def body(t, carry):
            r = p * _NSC + t
            pltpu.async_copy(g_sp.at[lst_v.at[r]], val_v.at[r], sem)
            return carry
        lax.fori_loop(0, _NSC, body, 0)

    def drain(p):
        def body(t, carry):
            r = p * _NSC + t
            pltpu.make_async_copy(
                g_sp.at[lst_v.at[r]], val_v.at[r], sem).wait()
            return carry
        lax.fori_loop(0, _NSC, body, 0)

    def accum(q, p):
        def body(g, carry):
            for t in range(_NSC):
                b0 = q * _QB + g * _L
                v = val_v[p * _NSC + t, pl.ds(g * _L, _L)]
                s_v[t, pl.ds(b0, _L)] = s_v[t, pl.ds(b0, _L)] + v
            return carry
        lax.fori_loop(0, _QB // _L, body, 0)

    def loss(q, p):
        def body(g, acc):
            for t in range(_NSC):
                b0 = q * _QB + g * _L
                v = val_v[p * _NSC + t, pl.ds(g * _L, _L)]
                sv = (s_v[t, pl.ds(b0, _L)] + v) * (1.0 / _C)
                wexp = jnp.exp(-jnp.abs(sv))
                lp = jnp.float32(_LP[6])
                for a in (_LP[5], _LP[4], _LP[3], _LP[2], _LP[1], _LP[0]):
                    lp = lp * wexp + jnp.float32(a)
                sp = jnp.maximum(sv, 0.0) + lp
                if t == 0:
                    acc = acc + (sv - sp)
                else:
                    acc = acc - sp
            return acc
        part = lax.fori_loop(0, _QB // _L, body, zeros)
        acc_v[...] = acc_v[...] + part

    nsp = _C * nq

    build(0, 0, 0)

    @pl.when(sid == 0)
    def _():
        gdesc.wait()

    plsc.subcore_barrier()

    fire(0)

    def sp_body(i, carry):
        build(i // nq, i % nq, i % 2)
        fire(i % 2)
        drain(1 - i % 2)
        accum((i - 1) % nq, 1 - i % 2)
        return carry

    lax.fori_loop(1, nsp - nq + 1, sp_body, 0)

    for i in range(nsp - nq + 1, nsp):
        p = i % 2
        build(_C - 1, i % nq, p)
        fire(p)
        drain(1 - p)
        loss((i - 1) % nq, 1 - p)
    p_last = (nsp - 1) % 2
    drain(p_last)
    loss(nq - 1, p_last)

    pltpu.sync_copy(acc_v, out_hbm.at[w])


def kernel(x, target, neg_samples, U):
    B = x.shape[0]
    bpw = B // _NW

    gram = pl.pallas_call(
        _tc_gram,
        grid=(1,),
        in_specs=[pl.BlockSpec((_VOC, _EMB), lambda i: (0, 0))],
        out_specs=pl.BlockSpec((_VOC, _VOC), lambda i: (0, 0)),
        out_shape=jax.ShapeDtypeStruct((_VOC, _VOC), jnp.float32),
    )(U)

    mesh = plsc.VectorSubcoreMesh(core_axis_name="c", subcore_axis_name="s")
    partials = pl.kernel(
        _sc_vals,
        out_type=jax.ShapeDtypeStruct((_NW, _L), jnp.float32),
        mesh=mesh,
        scratch_types=[
            pltpu.VMEM_SHARED((_VOC * _VOC,), jnp.float32),
            pltpu.VMEM((bpw * _C,), jnp.int32),
            pltpu.VMEM((bpw,), jnp.int32),
            pltpu.VMEM((bpw * _K,), jnp.int32),
            pltpu.VMEM((2 * _NSC, _QB), jnp.int32),
            pltpu.VMEM((2 * _NSC, _QB), jnp.float32),
            pltpu.VMEM((_NSC, bpw), jnp.float32),
            pltpu.VMEM((_L,), jnp.float32),
            pltpu.SemaphoreType.DMA,
            pltpu.SemaphoreType.DMA,
        ],
        compiler_params=pltpu.CompilerParams(needs_layout_passes=False),
    )(x.reshape(-1).astype(jnp.int32), target.astype(jnp.int32),
      neg_samples.reshape(-1).astype(jnp.int32), gram.reshape(-1))

    return -jnp.sum(partials)


# speedup vs baseline: 1.1583x; 1.1583x over previous
import jax
import jax.numpy as jnp
from jax import lax
from jax.experimental import pallas as pl
from jax.experimental.pallas import tpu as pltpu
from jax.experimental.pallas import tpu_sc as plsc

_VOC = 1000
_EMB = 64
_C = 4
_K = 20
_NIDX = _C + 1 + _K
_NSC = _K + 1
_NW = 32
_L = 16
_GCH = 128
_FK = 12
_LP = (3.507552052950621e-06, 0.9997924357286277, -0.4969779111678143,
       0.31459053537160714, -0.18878267362211323, 0.08172680837613401,
       -0.01720806112146555)


def _tc_gram(u_ref, g_ref):
    U = u_ref[...]
    g_ref[...] = lax.dot_general(
        U, U, (((1,), (1,)), ((), ())),
        preferred_element_type=jnp.float32,
        precision=jax.lax.Precision.HIGHEST)


def _sc_vals(idx_hbm, g_hbm, out_hbm, g_sp, idx_v, lst_v, val_v, s_v, acc_v, sem):
    bpw = idx_v.shape[1]
    cid = lax.axis_index("c")
    sid = lax.axis_index("s")
    w = sid * 2 + cid

    pltpu.sync_copy(idx_hbm.at[w], idx_v)

    @pl.when(sid == 0)
    def _():
        pltpu.sync_copy(g_hbm, g_sp)

    barriered = False
    for c in range(_C):
        def lst_body(g, carry):
            b0 = g * _L
            xc = idx_v[c, pl.ds(b0, _L)]
            for t in range(_NSC):
                tv = idx_v[_C + t, pl.ds(b0, _L)] * _VOC
                lst_v[pl.ds(t * bpw + b0, _L)] = tv + xc
            return carry

        lax.fori_loop(0, bpw // _L, lst_body, 0)

        if not barriered:
            plsc.subcore_barrier()
            barriered = True

        nch = _NSC * bpw // _GCH
        def fire_body(i, carry):
            descs = []
            for k in range(_FK):
                o = (i * _FK + k) * _GCH
                descs.append(pltpu.async_copy(
                    g_sp.at[lst_v.at[pl.ds(o, _GCH)]],
                    val_v.at[pl.ds(o, _GCH)], sem))
            for d in descs:
                d.wait()
            return carry

        lax.fori_loop(0, nch // _FK, fire_body, 0)

        if c < _C - 1:
            def sum_body(g, carry):
                b0 = g * _L
                for t in range(_NSC):
                    v = val_v[pl.ds(t * bpw + b0, _L)]
                    if c == 0:
                        s_v[t, pl.ds(b0, _L)] = v
                    else:
                        s_v[t, pl.ds(b0, _L)] = s_v[t, pl.ds(b0, _L)] + v
                return carry

            lax.fori_loop(0, bpw // _L, sum_body, 0)
        else:
            def loss_body(g, acc):
                b0 = g * _L
                for t in range(_NSC):
                    v = val_v[pl.ds(t * bpw + b0, _L)]
                    sv = (s_v[t, pl.ds(b0, _L)] + v) * (1.0 / _C)
                    wexp = jnp.exp(-jnp.abs(sv))
                    lp = jnp.float32(_LP[6])
                    for a in (_LP[5], _LP[4], _LP[3], _LP[2], _LP[1],
                              _LP[0]):
                        lp = lp * wexp + jnp.float32(a)
                    sp = jnp.maximum(sv, 0.0) + lp
                    if t == 0:
                        acc = acc + (sv - sp)
                    else:
                        acc = acc - sp
                return acc

            acc = lax.fori_loop(0, bpw // _L, loss_body,
                                jnp.zeros((_L,), jnp.float32))
            acc_v[...] = acc

    pltpu.sync_copy(acc_v, out_hbm.at[w])


def kernel(x, target, neg_samples, U):
    B = x.shape[0]
    bpw = B // _NW

    idx_all = jnp.concatenate(
        [x.T, target[None, :], neg_samples.T], axis=0).astype(jnp.int32)
    idx_blk = idx_all.reshape(_NIDX, _NW, bpw).transpose(1, 0, 2)

    gram = pl.pallas_call(
        _tc_gram,
        grid=(1,),
        in_specs=[pl.BlockSpec((_VOC, _EMB), lambda i: (0, 0))],
        out_specs=pl.BlockSpec((_VOC, _VOC), lambda i: (0, 0)),
        out_shape=jax.ShapeDtypeStruct((_VOC, _VOC), jnp.float32),
    )(U)

    mesh = plsc.VectorSubcoreMesh(core_axis_name="c", subcore_axis_name="s")
    partials = pl.kernel(
        _sc_vals,
        out_type=jax.ShapeDtypeStruct((_NW, _L), jnp.float32),
        mesh=mesh,
        scratch_types=[
            pltpu.VMEM_SHARED((_VOC * _VOC,), jnp.float32),
            pltpu.VMEM((_NIDX, bpw), jnp.int32),
            pltpu.VMEM((_NSC * bpw,), jnp.int32),
            pltpu.VMEM((_NSC * bpw,), jnp.float32),
            pltpu.VMEM((_NSC, bpw), jnp.float32),
            pltpu.VMEM((_L,), jnp.float32),
            pltpu.SemaphoreType.DMA,
        ],
        compiler_params=pltpu.CompilerParams(needs_layout_passes=False),
    )(idx_blk, gram.reshape(-1))

    return -jnp.sum(partials)


# speedup vs baseline: 1.1694x; 1.0096x over previous
import jax
import jax.numpy as jnp
from jax import lax
from jax.experimental import pallas as pl
from jax.experimental.pallas import tpu as pltpu
from jax.experimental.pallas import tpu_sc as plsc

_VOC = 1000
_EMB = 64
_C = 4
_K = 20
_NIDX = _C + 1 + _K
_NSC = _K + 1
_NW = 32
_L = 16
_GCH = 128
_FK = 28
_LP = (3.507552052950621e-06, 0.9997924357286277, -0.4969779111678143,
       0.31459053537160714, -0.18878267362211323, 0.08172680837613401,
       -0.01720806112146555)


def _tc_gram(u_ref, g_ref):
    U = u_ref[...]
    g_ref[...] = lax.dot_general(
        U, U, (((1,), (1,)), ((), ())),
        preferred_element_type=jnp.float32,
        precision=jax.lax.Precision.HIGHEST)


def _sc_vals(idx_hbm, g_hbm, out_hbm, g_sp, idx_v, lst_v, val_v, s_v, acc_v, sem):
    bpw = idx_v.shape[1]
    cid = lax.axis_index("c")
    sid = lax.axis_index("s")
    w = sid * 2 + cid

    pltpu.sync_copy(idx_hbm.at[w], idx_v)

    @pl.when(sid == 0)
    def _():
        pltpu.sync_copy(g_hbm, g_sp)

    barriered = False
    for c in range(_C):
        def lst_body(g, carry):
            b0 = g * _L
            xc = idx_v[c, pl.ds(b0, _L)]
            for t in range(_NSC):
                tv = idx_v[_C + t, pl.ds(b0, _L)] * _VOC
                lst_v[pl.ds(t * bpw + b0, _L)] = tv + xc
            return carry

        lax.fori_loop(0, bpw // _L, lst_body, 0)

        if not barriered:
            plsc.subcore_barrier()
            barriered = True

        nch = _NSC * bpw // _GCH
        def fire_body(i, carry):
            descs = []
            for k in range(_FK):
                o = (i * _FK + k) * _GCH
                descs.append(pltpu.async_copy(
                    g_sp.at[lst_v.at[pl.ds(o, _GCH)]],
                    val_v.at[pl.ds(o, _GCH)], sem))
            for d in descs:
                d.wait()
            return carry

        lax.fori_loop(0, nch // _FK, fire_body, 0)

        if c < _C - 1:
            def sum_body(g, carry):
                b0 = g * _L
                for t in range(_NSC):
                    v = val_v[pl.ds(t * bpw + b0, _L)]
                    if c == 0:
                        s_v[t, pl.ds(b0, _L)] = v
                    else:
                        s_v[t, pl.ds(b0, _L)] = s_v[t, pl.ds(b0, _L)] + v
                return carry

            lax.fori_loop(0, bpw // _L, sum_body, 0)
        else:
            def loss_body(g, acc):
                b0 = g * _L
                for t in range(_NSC):
                    v = val_v[pl.ds(t * bpw + b0, _L)]
                    sv = (s_v[t, pl.ds(b0, _L)] + v) * (1.0 / _C)
                    wexp = jnp.exp(-jnp.abs(sv))
                    lp = jnp.float32(_LP[6])
                    for a in (_LP[5], _LP[4], _LP[3], _LP[2], _LP[1],
                              _LP[0]):
                        lp = lp * wexp + jnp.float32(a)
                    sp = jnp.maximum(sv, 0.0) + lp
                    if t == 0:
                        acc = acc + (sv - sp)
                    else:
                        acc = acc - sp
                return acc

            acc = lax.fori_loop(0, bpw // _L, loss_body,
                                jnp.zeros((_L,), jnp.float32))
            acc_v[...] = acc

    pltpu.sync_copy(acc_v, out_hbm.at[w])


def kernel(x, target, neg_samples, U):
    B = x.shape[0]
    bpw = B // _NW

    idx_all = jnp.concatenate(
        [x.T, target[None, :], neg_samples.T], axis=0).astype(jnp.int32)
    idx_blk = idx_all.reshape(_NIDX, _NW, bpw).transpose(1, 0, 2)

    gram = pl.pallas_call(
        _tc_gram,
        grid=(1,),
        in_specs=[pl.BlockSpec((_VOC, _EMB), lambda i: (0, 0))],
        out_specs=pl.BlockSpec((_VOC, _VOC), lambda i: (0, 0)),
        out_shape=jax.ShapeDtypeStruct((_VOC, _VOC), jnp.float32),
    )(U)

    mesh = plsc.VectorSubcoreMesh(core_axis_name="c", subcore_axis_name="s")
    partials = pl.kernel(
        _sc_vals,
        out_type=jax.ShapeDtypeStruct((_NW, _L), jnp.float32),
        mesh=mesh,
        scratch_types=[
            pltpu.VMEM_SHARED((_VOC * _VOC,), jnp.float32),
            pltpu.VMEM((_NIDX, bpw), jnp.int32),
            pltpu.VMEM((_NSC * bpw,), jnp.int32),
            pltpu.VMEM((_NSC * bpw,), jnp.float32),
            pltpu.VMEM((_NSC, bpw), jnp.float32),
            pltpu.VMEM((_L,), jnp.float32),
            pltpu.SemaphoreType.DMA,
        ],
        compiler_params=pltpu.CompilerParams(needs_layout_passes=False),
    )(idx_blk, gram.reshape(-1))

    return -jnp.sum(partials)


# speedup vs baseline: 1.1780x; 1.0073x over previous
import jax
import jax.numpy as jnp
from jax import lax
from jax.experimental import pallas as pl
from jax.experimental.pallas import tpu as pltpu
from jax.experimental.pallas import tpu_sc as plsc

_VOC = 1000
_EMB = 64
_C = 4
_K = 20
_NIDX = _C + 1 + _K
_NSC = _K + 1
_NW = 32
_L = 16
_GCH = 128
_FK = 84
_LP = (3.507552052950621e-06, 0.9997924357286277, -0.4969779111678143,
       0.31459053537160714, -0.18878267362211323, 0.08172680837613401,
       -0.01720806112146555)


def _tc_gram(u_ref, g_ref):
    U = u_ref[...]
    g_ref[...] = lax.dot_general(
        U, U, (((1,), (1,)), ((), ())),
        preferred_element_type=jnp.float32,
        precision=jax.lax.Precision.HIGHEST)


def _sc_vals(idx_hbm, g_hbm, out_hbm, g_sp, idx_v, lst_v, val_v, s_v, acc_v, sem):
    bpw = idx_v.shape[1]
    cid = lax.axis_index("c")
    sid = lax.axis_index("s")
    w = sid * 2 + cid

    pltpu.sync_copy(idx_hbm.at[w], idx_v)

    @pl.when(sid == 0)
    def _():
        pltpu.sync_copy(g_hbm, g_sp)

    barriered = False
    for c in range(_C):
        def lst_body(g, carry):
            b0 = g * _L
            xc = idx_v[c, pl.ds(b0, _L)]
            for t in range(_NSC):
                tv = idx_v[_C + t, pl.ds(b0, _L)] * _VOC
                lst_v[pl.ds(t * bpw + b0, _L)] = tv + xc
            return carry

        lax.fori_loop(0, bpw // _L, lst_body, 0)

        if not barriered:
            plsc.subcore_barrier()
            barriered = True

        nch = _NSC * bpw // _GCH
        def fire_body(i, carry):
            descs = []
            for k in range(_FK):
                o = (i * _FK + k) * _GCH
                descs.append(pltpu.async_copy(
                    g_sp.at[lst_v.at[pl.ds(o, _GCH)]],
                    val_v.at[pl.ds(o, _GCH)], sem))
            for d in descs:
                d.wait()
            return carry

        lax.fori_loop(0, nch // _FK, fire_body, 0)

        if c < _C - 1:
            def sum_body(g, carry):
                b0 = g * _L
                for t in range(_NSC):
                    v = val_v[pl.ds(t * bpw + b0, _L)]
                    if c == 0:
                        s_v[t, pl.ds(b0, _L)] = v
                    else:
                        s_v[t, pl.ds(b0, _L)] = s_v[t, pl.ds(b0, _L)] + v
                return carry

            lax.fori_loop(0, bpw // _L, sum_body, 0)
        else:
            def loss_body(g, acc):
                b0 = g * _L
                for t in range(_NSC):
                    v = val_v[pl.ds(t * bpw + b0, _L)]
                    sv = (s_v[t, pl.ds(b0, _L)] + v) * (1.0 / _C)
                    wexp = jnp.exp(-jnp.abs(sv))
                    lp = jnp.float32(_LP[6])
                    for a in (_LP[5], _LP[4], _LP[3], _LP[2], _LP[1],
                              _LP[0]):
                        lp = lp * wexp + jnp.float32(a)
                    sp = jnp.maximum(sv, 0.0) + lp
                    if t == 0:
                        acc = acc + (sv - sp)
                    else:
                        acc = acc - sp
                return acc

            acc = lax.fori_loop(0, bpw // _L, loss_body,
                                jnp.zeros((_L,), jnp.float32))
            acc_v[...] = acc

    pltpu.sync_copy(acc_v, out_hbm.at[w])


def kernel(x, target, neg_samples, U):
    B = x.shape[0]
    bpw = B // _NW

    idx_all = jnp.concatenate(
        [x.T, target[None, :], neg_samples.T], axis=0).astype(jnp.int32)
    idx_blk = idx_all.reshape(_NIDX, _NW, bpw).transpose(1, 0, 2)

    gram = pl.pallas_call(
        _tc_gram,
        grid=(1,),
        in_specs=[pl.BlockSpec((_VOC, _EMB), lambda i: (0, 0))],
        out_specs=pl.BlockSpec((_VOC, _VOC), lambda i: (0, 0)),
        out_shape=jax.ShapeDtypeStruct((_VOC, _VOC), jnp.float32),
    )(U)

    mesh = plsc.VectorSubcoreMesh(core_axis_name="c", subcore_axis_name="s")
    partials = pl.kernel(
        _sc_vals,
        out_type=jax.ShapeDtypeStruct((_NW, _L), jnp.float32),
        mesh=mesh,
        scratch_types=[
            pltpu.VMEM_SHARED((_VOC * _VOC,), jnp.float32),
            pltpu.VMEM((_NIDX, bpw), jnp.int32),
            pltpu.VMEM((_NSC * bpw,), jnp.int32),
            pltpu.VMEM((_NSC * bpw,), jnp.float32),
            pltpu.VMEM((_NSC, bpw), jnp.float32),
            pltpu.VMEM((_L,), jnp.float32),
            pltpu.SemaphoreType.DMA,
        ],
        compiler_params=pltpu.CompilerParams(needs_layout_passes=False),
    )(idx_blk, gram.reshape(-1))

    return -jnp.sum(partials)
